# baseline (speedup 1.0000x reference)
import functools

import jax
import jax.numpy as jnp
from jax import lax
from jax.experimental import pallas as pl
from jax.experimental.pallas import tpu as pltpu
from jax.experimental.pallas import tpu_sc as plsc

HD = 128
E = 100000
NTILES = 32
EDG = 3328
EPAD = NTILES * EDG
K = 128
NB = EDG // K
ZB = 32
F32 = jnp.float32


def _sc_ln_rows(xj, pr, gv, bv, nrows):
    lanes = lax.iota(jnp.int32, 16)

    def row_body(r, _):
        d = [xj[r, pl.ds(16 * i, 16)] - pr[r, pl.ds(16 * i, 16)]
             for i in range(8)]
        s = ((d[0] + d[1]) + (d[2] + d[3])) + ((d[4] + d[5]) + (d[6] + d[7]))
        q = [di * di for di in d]
        sq = ((q[0] + q[1]) + (q[2] + q[3])) + ((q[4] + q[5]) + (q[6] + q[7]))
        for shf in (8, 4, 2, 1):
            idx = lanes ^ shf
            s = s + s.at[idx].get(mode='promise_in_bounds')
            sq = sq + sq.at[idx].get(mode='promise_in_bounds')
        mv = s * jnp.float32(1.0 / HD)
        var = sq * jnp.float32(1.0 / HD) - mv * mv
        xv = var + jnp.float32(1e-5)
        iv = lax.bitcast_convert_type(xv, jnp.int32)
        yv = lax.bitcast_convert_type(
            jnp.int32(0x5F3759DF) - lax.shift_right_logical(iv, jnp.int32(1)),
            F32)
        for _ in range(3):
            yv = yv * (jnp.float32(1.5) - jnp.float32(0.5) * xv * yv * yv)
        for i in range(8):
            xj[r, pl.ds(16 * i, 16)] = (d[i] - mv) * yv * gv[i] + bv[i]
        return 0
    lax.fori_loop(0, nrows, row_body, 0)


def _sc_segment_call(mode, xsrc, pred, gvec, bvec, src, dst, C, R):
    has_ln = mode in ('gather_ln', 'msg')
    scatter = mode != 'msg'
    gathers = mode in ('gather', 'gather_ln', 'msg')
    const_rows = mode == 'count'
    KB = 64 if mode == 'gather_ln' else K
    NBM = EDG // KB
    JG = KB // 16
    SR = R // 16
    mesh = plsc.VectorSubcoreMesh(core_axis_name="c", subcore_axis_name="s")

    scratch = []
    for _p in range(2):
        scratch.append(pltpu.VMEM((1, KB), jnp.int32))
        scratch.append(pltpu.VMEM((KB,), jnp.int32))
        if not const_rows:
            scratch.append(pltpu.VMEM((KB, HD), F32))
        if gathers:
            scratch.append(pltpu.VMEM((KB,), jnp.int32))
        if has_ln:
            scratch.append(pltpu.VMEM((KB,), jnp.int32))
            scratch.append(pltpu.VMEM((KB, HD), F32))
    if const_rows:
        scratch.append(pltpu.VMEM((KB, HD), F32))
    if has_ln:
        scratch.append(pltpu.VMEM((HD,), F32))
        scratch.append(pltpu.VMEM((HD,), F32))
    for _p in range(2):
        scratch.append(pltpu.SemaphoreType.DMA)
        scratch.append(pltpu.SemaphoreType.DMA)
        scratch.append(pltpu.SemaphoreType.DMA)
    if scatter:
        scratch.append(pltpu.VMEM_SHARED((R + ZB, HD), F32))

    def body(*refs):
        it = iter(refs)
        xsrc_h = next(it) if not const_rows else None
        pred_h = next(it) if has_ln else None
        g_h = next(it) if has_ln else None
        b_h = next(it) if has_ln else None
        src_h = next(it) if gathers else None
        dst_h = next(it)
        if const_rows:
            ones_h = next(it)
        if scatter:
            zb_h = next(it)
            sums_h = next(it)
        else:
            msg_h = next(it)
        dlb = [None, None]; didx = [None, None]; xjs = [None, None]
        sidx = [None, None]; pidx = [None, None]; prs = [None, None]
        for _p in range(2):
            dlb[_p] = next(it)
            didx[_p] = next(it)
            if not const_rows:
                xjs[_p] = next(it)
            if gathers:
                sidx[_p] = next(it)
            if has_ln:
                pidx[_p] = next(it)
                prs[_p] = next(it)
        if const_rows:
            xc = next(it)
            xjs = [xc, xc]
        if has_ln:
            gbuf = next(it); bbuf = next(it)
        sem_ld = [None, None]; sem_g = [None, None]; sem_s = [None, None]
        for _p in range(2):
            sem_ld[_p] = next(it); sem_g[_p] = next(it); sem_s[_p] = next(it)
        if scatter:
            aggr_sh = next(it)

        cid = lax.axis_index("c")
        sid = lax.axis_index("s")
        w = cid * 16 + sid
        base_e = w * EDG
        if has_ln:
            pltpu.sync_copy(g_h, gbuf)
            pltpu.sync_copy(b_h, bbuf)
            gv = [gbuf[pl.ds(16 * i, 16)] for i in range(8)]
            bv = [bbuf[pl.ds(16 * i, 16)] for i in range(8)]
        if const_rows:
            pltpu.sync_copy(ones_h, xjs[0])

        def issue_loads(pp, bi):
            ebg = base_e + bi * KB
            pltpu.async_copy(dst_h.at[pl.ds(ebg, KB)], didx[pp], sem_ld[pp])
            if gathers:
                pltpu.async_copy(src_h.at[pl.ds(ebg, KB)], sidx[pp],
                                 sem_ld[pp])
            if mode == 'linear':
                pltpu.async_copy(xsrc_h.at[pl.ds(ebg, KB)], xjs[pp],
                                 sem_ld[pp])

        def wait_loads(pp):
            pltpu.make_async_copy(dst_h.at[pl.ds(0, KB)], didx[pp],
                                  sem_ld[pp]).wait()
            if gathers:
                pltpu.make_async_copy(src_h.at[pl.ds(0, KB)], sidx[pp],
                                      sem_ld[pp]).wait()
            if mode == 'linear':
                pltpu.make_async_copy(xsrc_h.at[pl.ds(0, KB)], xjs[pp],
                                      sem_ld[pp]).wait()

        def issue_out(pp, bi):
            if scatter:
                pltpu.async_copy(xjs[pp], aggr_sh.at[dlb[pp].at[0]],
                                 sem_s[pp], add=True)
            else:
                ebg = base_e + bi * KB
                pltpu.async_copy(xjs[pp], msg_h.at[pl.ds(ebg, KB)],
                                 sem_s[pp])

        def wait_out(pp):
            if scatter:
                pltpu.make_async_copy(xjs[pp], aggr_sh.at[dlb[pp].at[0]],
                                      sem_s[pp]).wait()
            else:
                pltpu.make_async_copy(xjs[pp], msg_h.at[pl.ds(0, KB)],
                                      sem_s[pp]).wait()

        for ci in range(C):
            lo = ci * R
            if scatter:
                for n in range(SR // ZB):
                    off = sid * SR + n * ZB
                    pltpu.sync_copy(zb_h, aggr_sh.at[pl.ds(off, ZB)])

                @pl.when(sid == 0)
                def _zero_trash():
                    pltpu.sync_copy(zb_h, aggr_sh.at[pl.ds(R, ZB)])

                plsc.subcore_barrier()

            issue_loads(0, 0)

            def pair_body(it2, _):
                for sb in range(2):
                    bi = 2 * it2 + sb
                    A, B = sb, 1 - sb
                    wait_loads(A)
                    for j in range(JG):
                        dv = didx[A][pl.ds(16 * j, 16)]
                        if scatter:
                            m = (dv >= lo) & (dv < lo + R)
                            dlb[A][0, pl.ds(16 * j, 16)] = \
                                jnp.where(m, dv - lo, R)
                        if has_ln:
                            pidx[A][pl.ds(16 * j, 16)] = jnp.maximum(dv, 0)
                    if gathers:
                        cp1 = pltpu.async_copy(xsrc_h.at[sidx[A]], xjs[A],
                                               sem_g[A])
                        if has_ln:
                            cp2 = pltpu.async_copy(pred_h.at[pidx[A]],
                                                   prs[A], sem_g[A])

                    @pl.when(bi + 1 < NBM)
                    def _prefetch():
                        @pl.when(bi >= 1)
                        def _drain_b():
                            wait_out(B)
                        issue_loads(B, bi + 1)

                    if gathers:
                        cp1.wait()
                        if has_ln:
                            cp2.wait()
                            _sc_ln_rows(xjs[A], prs[A], gv, bv, KB)
                    issue_out(A, bi)
                return 0
            lax.fori_loop(0, NBM // 2, pair_body, 0)
            wait_out(0)
            wait_out(1)

            if scatter:
                plsc.subcore_barrier()
                cbase = cid * (C * R) + lo
                for n in range(SR // ZB):
                    off = sid * SR + n * ZB
                    pltpu.sync_copy(aggr_sh.at[pl.ds(off, ZB)],
                                    sums_h.at[pl.ds(cbase + off, ZB)])
                plsc.subcore_barrier()

    if scatter:
        out_type = jax.ShapeDtypeStruct((2 * C * R, HD), F32)
    else:
        out_type = jax.ShapeDtypeStruct((EPAD, HD), F32)
    kfn = pl.kernel(body, out_type=out_type, mesh=mesh, scratch_types=scratch)
    args = []
    if not const_rows:
        args.append(xsrc)
    if has_ln:
        args += [pred, gvec, bvec]
    if gathers:
        args.append(src)
    args.append(dst)
    if const_rows:
        args.append(jnp.pad(jnp.ones((KB, 1), F32), ((0, 0), (0, HD - 1))))
    if scatter:
        args.append(jnp.zeros((ZB, HD), F32))
        return kfn(*args).reshape(2, C * R, HD)
    return kfn(*args)


def _full(shape):
    return pl.BlockSpec(shape, lambda i: tuple(0 for _ in shape))


def _mlp2_call(x, w1t, b1, w2t, b2, B=1000):
    N = x.shape[0]

    def body(x_ref, w1_ref, b1_ref, w2_ref, b2_ref, o_ref):
        h = jnp.dot(x_ref[...], w1_ref[...], preferred_element_type=F32)
        h = jnp.maximum(h + b1_ref[...], 0.0)
        o_ref[...] = jnp.dot(h, w2_ref[...],
                             preferred_element_type=F32) + b2_ref[...]

    return pl.pallas_call(
        body,
        grid=(N // B,),
        in_specs=[pl.BlockSpec((B, HD), lambda i: (i, 0)),
                  _full((HD, HD)), _full((1, HD)),
                  _full((HD, HD)), _full((1, HD))],
        out_specs=pl.BlockSpec((B, HD), lambda i: (i, 0)),
        out_shape=jax.ShapeDtypeStruct((N, HD), F32),
    )(x, w1t, b1[None], w2t, b2[None])


def _prmp_combine_call(x_dst, sums, cnts, wxt, wat, b, B=1000):
    N = x_dst.shape[0]
    G = N // B

    def body(x_ref, s0, s1, c0, c1, wx, wa, b_ref, o_ref, ps, pq):
        cnt = jnp.maximum(c0[0][:, :1] + c1[0][:, :1], 1.0)
        aggr = (s0[0] + s1[0]) / cnt
        y = jnp.dot(x_ref[...], wx[...], preferred_element_type=F32)
        y = y + jnp.dot(aggr, wa[...], preferred_element_type=F32) + b_ref[...]
        o_ref[...] = y
        ps[...] = jnp.sum(y.reshape(8, B // 8, HD), axis=1)[None]
        pq[...] = jnp.sum((y * y).reshape(8, B // 8, HD), axis=1)[None]

    sspec0 = pl.BlockSpec((1, B, HD), lambda i: (0, i, 0))
    sspec1 = pl.BlockSpec((1, B, HD), lambda i: (1, i, 0))
    cspec0 = pl.BlockSpec((1, B, HD), lambda i: (0, i, 0))
    cspec1 = pl.BlockSpec((1, B, HD), lambda i: (1, i, 0))
    return pl.pallas_call(
        body,
        grid=(G,),
        in_specs=[pl.BlockSpec((B, HD), lambda i: (i, 0)),
                  sspec0, sspec1, cspec0, cspec1,
                  _full((HD, HD)), _full((HD, HD)), _full((1, HD))],
        out_specs=[pl.BlockSpec((B, HD), lambda i: (i, 0)),
                   pl.BlockSpec((1, 8, HD), lambda i: (i, 0, 0)),
                   pl.BlockSpec((1, 8, HD), lambda i: (i, 0, 0))],
        out_shape=[jax.ShapeDtypeStruct((N, HD), F32),
                   jax.ShapeDtypeStruct((G, 8, HD), F32),
                   jax.ShapeDtypeStruct((G, 8, HD), F32)],
    )(x_dst, sums, sums, cnts, cnts, wxt, wat, b[None])


def _sage_combine_call(x_rev, sums_h, cnts_h, sums_w, cnts_w,
                       wlh_t, wlw_t, wrs_t, bls, nh, nw, B=1000):
    N = x_rev.shape[0]
    G = N // B
    gh, gw = nh // B, nw // B

    def body(x_ref, s0h, s1h, c0h, c1h, s0w, s1w, c0w, c1w,
             wlh, wlw, wrs, b_ref, o_ref, ps, pq):
        i = pl.program_id(0)
        y = jnp.dot(x_ref[...], wrs[...], preferred_element_type=F32)
        o_ref[...] = y + b_ref[...]

        @pl.when(i < gh)
        def _add_has():
            cnt = jnp.maximum(c0h[0][:, :1] + c1h[0][:, :1], 1.0)
            aggr = (s0h[0] + s1h[0]) / cnt
            o_ref[...] += jnp.dot(aggr, wlh[...], preferred_element_type=F32)

        @pl.when(i < gw)
        def _add_wrote():
            cnt = jnp.maximum(c0w[0][:, :1] + c1w[0][:, :1], 1.0)
            aggr = (s0w[0] + s1w[0]) / cnt
            o_ref[...] += jnp.dot(aggr, wlw[...], preferred_element_type=F32)

        t = o_ref[...]
        ps[...] = jnp.sum(t.reshape(8, B // 8, HD), axis=1)[None]
        pq[...] = jnp.sum((t * t).reshape(8, B // 8, HD), axis=1)[None]

    def clamp_map(g, lead):
        return lambda i: (lead, jnp.minimum(i, g - 1), 0)

    return pl.pallas_call(
        body,
        grid=(G,),
        in_specs=[pl.BlockSpec((B, HD), lambda i: (i, 0)),
                  pl.BlockSpec((1, B, HD), clamp_map(gh, 0)),
                  pl.BlockSpec((1, B, HD), clamp_map(gh, 1)),
                  pl.BlockSpec((1, B, HD), clamp_map(gh, 0)),
                  pl.BlockSpec((1, B, HD), clamp_map(gh, 1)),
                  pl.BlockSpec((1, B, HD), clamp_map(gw, 0)),
                  pl.BlockSpec((1, B, HD), clamp_map(gw, 1)),
                  pl.BlockSpec((1, B, HD), clamp_map(gw, 0)),
                  pl.BlockSpec((1, B, HD), clamp_map(gw, 1)),
                  _full((HD, HD)), _full((HD, HD)), _full((HD, HD)),
                  _full((1, HD))],
        out_specs=[pl.BlockSpec((B, HD), lambda i: (i, 0)),
                   pl.BlockSpec((1, 8, HD), lambda i: (i, 0, 0)),
                   pl.BlockSpec((1, 8, HD), lambda i: (i, 0, 0))],
        out_shape=[jax.ShapeDtypeStruct((N, HD), F32),
                   jax.ShapeDtypeStruct((G, 8, HD), F32),
                   jax.ShapeDtypeStruct((G, 8, HD), F32)],
    )(x_rev, sums_h, sums_h, cnts_h, cnts_h, sums_w, sums_w, cnts_w, cnts_w,
      wlh_t, wlw_t, wrs_t, bls[None])


def _ln_apply_call(y, ps, pq, g, b, resid, B=1000):
    N = y.shape[0]
    G = ps.shape[0]
    cnt = float(N * HD)

    def body(*refs):
        if resid is not None:
            y_ref, ps_ref, pq_ref, g_ref, b_ref, r_ref, o_ref = refs
        else:
            y_ref, ps_ref, pq_ref, g_ref, b_ref, o_ref = refs
        tot = jnp.sum(ps_ref[...])
        tsq = jnp.sum(pq_ref[...])
        mu = tot / cnt
        var = tsq / cnt - mu * mu
        denom = jnp.sqrt(jnp.maximum(var, 0.0)) + 1e-5
        out = jnp.maximum((y_ref[...] - mu) / denom * g_ref[...] + b_ref[...],
                          0.0)
        if resid is not None:
            out = out + r_ref[...]
        o_ref[...] = out

    in_specs = [pl.BlockSpec((B, HD), lambda i: (i, 0)),
                _full((G, 8, HD)), _full((G, 8, HD)),
                _full((1, HD)), _full((1, HD))]
    args = [y, ps, pq, g[None], b[None]]
    if resid is not None:
        in_specs.append(pl.BlockSpec((B, HD), lambda i: (i, 0)))
        args.append(resid)
    return pl.pallas_call(
        body,
        grid=(N // B,),
        in_specs=in_specs,
        out_specs=pl.BlockSpec((B, HD), lambda i: (i, 0)),
        out_shape=jax.ShapeDtypeStruct((N, HD), F32),
    )(*args)


def _head_call(x, w1t, b1, w2, b2, B=1000):
    N = x.shape[0]

    def body(x_ref, w1_ref, b1_ref, w2_ref, b2_ref, o_ref):
        h = jnp.dot(x_ref[...], w1_ref[...], preferred_element_type=F32)
        h = jnp.maximum(h + b1_ref[...], 0.0)
        o_ref[...] = jnp.dot(h, w2_ref[...],
                             preferred_element_type=F32) + b2_ref[...]

    return pl.pallas_call(
        body,
        grid=(N // B,),
        in_specs=[pl.BlockSpec((B, HD), lambda i: (i, 0)),
                  _full((HD, 64)), _full((1, 64)),
                  _full((64, 1)), _full((1, 1))],
        out_specs=pl.BlockSpec((B, 1), lambda i: (i, 0)),
        out_shape=jax.ShapeDtypeStruct((N, 1), F32),
    )(x, w1t, b1[None], w2, b2[None])


def _pad_edges(ei):
    pad = EPAD - E
    src = jnp.concatenate([ei[0], jnp.zeros((pad,), jnp.int32)])
    dst = jnp.concatenate([ei[1], jnp.full((pad,), -1, jnp.int32)])
    return src, dst


def _mlp2_p(p, x):
    return _mlp2_call(x, p['l1']['W'].T, p['l1']['b'],
                      p['l2']['W'].T, p['l2']['b'])


def kernel(params, x_product, x_customer, x_review, ei_of_product,
           ei_has_review, ei_by_customer, ei_wrote_review):
    p = params
    hp = _mlp2_p(p['enc_product'], x_product)
    hc = _mlp2_p(p['enc_customer'], x_customer)
    hr = _mlp2_p(p['enc_review'], x_review)

    src_op, dst_op = _pad_edges(ei_of_product)
    src_hr, dst_hr = _pad_edges(ei_has_review)
    src_bc, dst_bc = _pad_edges(ei_by_customer)
    src_wr, dst_wr = _pad_edges(ei_wrote_review)

    RA = 10240
    CP, CC = 1, 5

    c_op = _sc_segment_call('count', None, None, None, None, None, dst_op,
                            CP, RA)
    c_bc = _sc_segment_call('count', None, None, None, None, None, dst_bc,
                            CC, RA)
    c_hr = _sc_segment_call('count', None, None, None, None, None, dst_hr,
                            CP, RA)
    c_wr = _sc_segment_call('count', None, None, None, None, None, dst_wr,
                            CC, RA)

    for li in range(2):
        L = p['layers'][li]
        pp, pc = L['prmp_product'], L['prmp_customer']
        pred_p = _mlp2_p(pp['pred'], hp)
        pred_c = _mlp2_p(pc['pred'], hc)

        s_pp = _sc_segment_call('gather_ln', hr, pred_p, pp['norm_g'],
                                pp['norm_b'], src_op, dst_op, CP, RA)
        msg_c = _sc_segment_call('msg', hr, pred_c, pc['norm_g'],
                                 pc['norm_b'], src_bc, dst_bc, 1, RA)
        s_pc = _sc_segment_call('linear', msg_c, None, None, None,
                                None, dst_bc, CC, RA)
        s_sh = _sc_segment_call('gather', hp, None, None, None,
                                src_hr, dst_hr, CP, RA)
        s_sw = _sc_segment_call('gather', hc, None, None, None,
                                src_wr, dst_wr, CC, RA)
        c_pp, c_pc, c_sh, c_sw = c_op, c_bc, c_hr, c_wr

        Wp = pp['update']['W']
        out_p, psp, pqp = _prmp_combine_call(
            hp, s_pp, c_pp, Wp[:, :HD].T, Wp[:, HD:].T, pp['update']['b'])
        Wc = pc['update']['W']
        out_c, psc, pqc = _prmp_combine_call(
            hc, s_pc, c_pc, Wc[:, :HD].T, Wc[:, HD:].T, pc['update']['b'])

        sh, sw = L['sage_has'], L['sage_wrote']
        out_r, psr, pqr = _sage_combine_call(
            hr, s_sh, c_sh, s_sw, c_sw,
            sh['Wl'].T, sw['Wl'].T, (sh['Wr'] + sw['Wr']).T,
            sh['bl'] + sw['bl'], 10000, 50000)

        nrm = L['norm']
        res = li > 0
        hp_n = _ln_apply_call(out_p, psp, pqp, nrm['product']['g'],
                              nrm['product']['b'], hp if res else None)
        hc_n = _ln_apply_call(out_c, psc, pqc, nrm['customer']['g'],
                              nrm['customer']['b'], hc if res else None)
        hr_n = _ln_apply_call(out_r, psr, pqr, nrm['review']['g'],
                              nrm['review']['b'], hr if res else None)
        hp, hc, hr = hp_n, hc_n, hr_n

    out = _head_call(hr, p['head1']['W'].T, p['head1']['b'],
                     p['head2']['W'].T, p['head2']['b'])
    return out[:, 0]


# speedup vs baseline: 2.0638x; 2.0638x over previous
import functools

import jax
import jax.numpy as jnp
from jax import lax
from jax.experimental import pallas as pl
from jax.experimental.pallas import tpu as pltpu
from jax.experimental.pallas import tpu_sc as plsc

HD = 128
E = 100000
NTILES = 32
EDG = 3328
EPAD = NTILES * EDG
K = 128
NB = EDG // K
ZB = 32
F32 = jnp.float32


def _sc_ln_rows(xj, pr, gv, bv, nrows):
    lanes = lax.iota(jnp.int32, 16)

    def row_body(r, _):
        d = [xj[r, pl.ds(16 * i, 16)] - pr[r, pl.ds(16 * i, 16)]
             for i in range(8)]
        s = ((d[0] + d[1]) + (d[2] + d[3])) + ((d[4] + d[5]) + (d[6] + d[7]))
        q = [di * di for di in d]
        sq = ((q[0] + q[1]) + (q[2] + q[3])) + ((q[4] + q[5]) + (q[6] + q[7]))
        for shf in (8, 4, 2, 1):
            idx = lanes ^ shf
            s = s + s.at[idx].get(mode='promise_in_bounds')
            sq = sq + sq.at[idx].get(mode='promise_in_bounds')
        mv = s * jnp.float32(1.0 / HD)
        var = sq * jnp.float32(1.0 / HD) - mv * mv
        xv = var + jnp.float32(1e-5)
        iv = lax.bitcast_convert_type(xv, jnp.int32)
        yv = lax.bitcast_convert_type(
            jnp.int32(0x5F3759DF) - lax.shift_right_logical(iv, jnp.int32(1)),
            F32)
        for _ in range(3):
            yv = yv * (jnp.float32(1.5) - jnp.float32(0.5) * xv * yv * yv)
        for i in range(8):
            xj[r, pl.ds(16 * i, 16)] = (d[i] - mv) * yv * gv[i] + bv[i]
        return 0
    lax.fori_loop(0, nrows, row_body, 0)


def _sc_segment_call(mode, xsrc, pred, gvec, bvec, src, dst, C, R):
    has_ln = mode in ('gather_ln', 'msg')
    scatter = mode != 'msg'
    gathers = mode in ('gather', 'gather_ln', 'msg')
    const_rows = mode == 'count'
    KB = 64 if mode == 'gather_ln' else K
    NBM = EDG // KB
    JG = KB // 16
    SR = R // 16
    mesh = plsc.VectorSubcoreMesh(core_axis_name="c", subcore_axis_name="s")

    scratch = []
    for _p in range(2):
        scratch.append(pltpu.VMEM((1, KB), jnp.int32))
        scratch.append(pltpu.VMEM((KB,), jnp.int32))
        if not const_rows:
            scratch.append(pltpu.VMEM((KB, HD), F32))
        if gathers:
            scratch.append(pltpu.VMEM((KB,), jnp.int32))
        if has_ln:
            scratch.append(pltpu.VMEM((KB,), jnp.int32))
            scratch.append(pltpu.VMEM((KB, HD), F32))
    if const_rows:
        scratch.append(pltpu.VMEM((KB, HD), F32))
    if has_ln:
        scratch.append(pltpu.VMEM((HD,), F32))
        scratch.append(pltpu.VMEM((HD,), F32))
    for _p in range(2):
        scratch.append(pltpu.SemaphoreType.DMA)
        scratch.append(pltpu.SemaphoreType.DMA)
        scratch.append(pltpu.SemaphoreType.DMA)
    if scatter:
        scratch.append(pltpu.VMEM_SHARED((R + ZB, HD), F32))

    def body(*refs):
        it = iter(refs)
        xsrc_h = next(it) if not const_rows else None
        pred_h = next(it) if has_ln else None
        g_h = next(it) if has_ln else None
        b_h = next(it) if has_ln else None
        src_h = next(it) if gathers else None
        dst_h = next(it)
        if const_rows:
            ones_h = next(it)
        if scatter:
            zb_h = next(it)
            sums_h = next(it)
        else:
            msg_h = next(it)
        dlb = [None, None]; didx = [None, None]; xjs = [None, None]
        sidx = [None, None]; pidx = [None, None]; prs = [None, None]
        for _p in range(2):
            dlb[_p] = next(it)
            didx[_p] = next(it)
            if not const_rows:
                xjs[_p] = next(it)
            if gathers:
                sidx[_p] = next(it)
            if has_ln:
                pidx[_p] = next(it)
                prs[_p] = next(it)
        if const_rows:
            xc = next(it)
            xjs = [xc, xc]
        if has_ln:
            gbuf = next(it); bbuf = next(it)
        sem_ld = [None, None]; sem_g = [None, None]; sem_s = [None, None]
        for _p in range(2):
            sem_ld[_p] = next(it); sem_g[_p] = next(it); sem_s[_p] = next(it)
        if scatter:
            aggr_sh = next(it)

        cid = lax.axis_index("c")
        sid = lax.axis_index("s")
        w = cid * 16 + sid
        base_e = w * EDG
        if has_ln:
            pltpu.sync_copy(g_h, gbuf)
            pltpu.sync_copy(b_h, bbuf)
            gv = [gbuf[pl.ds(16 * i, 16)] for i in range(8)]
            bv = [bbuf[pl.ds(16 * i, 16)] for i in range(8)]
        if const_rows:
            pltpu.sync_copy(ones_h, xjs[0])

        def issue_loads(pp, bi):
            ebg = base_e + bi * KB
            pltpu.async_copy(dst_h.at[pl.ds(ebg, KB)], didx[pp], sem_ld[pp])
            if gathers:
                pltpu.async_copy(src_h.at[pl.ds(ebg, KB)], sidx[pp],
                                 sem_ld[pp])
            if mode == 'linear':
                pltpu.async_copy(xsrc_h.at[pl.ds(ebg, KB)], xjs[pp],
                                 sem_ld[pp])

        def wait_loads(pp):
            pltpu.make_async_copy(dst_h.at[pl.ds(0, KB)], didx[pp],
                                  sem_ld[pp]).wait()
            if gathers:
                pltpu.make_async_copy(src_h.at[pl.ds(0, KB)], sidx[pp],
                                      sem_ld[pp]).wait()
            if mode == 'linear':
                pltpu.make_async_copy(xsrc_h.at[pl.ds(0, KB)], xjs[pp],
                                      sem_ld[pp]).wait()

        for ci in range(C):
            lo = ci * R
            if scatter:
                for n in range(SR // ZB):
                    off = sid * SR + n * ZB
                    pltpu.sync_copy(zb_h, aggr_sh.at[pl.ds(off, ZB)])

                @pl.when(sid == 0)
                def _zero_trash():
                    pltpu.sync_copy(zb_h, aggr_sh.at[pl.ds(R, ZB)])

                plsc.subcore_barrier()

            issue_loads(0, 0)

            def pair_body(it2, _):
                for sb in range(2):
                    bi = 2 * it2 + sb
                    A, B = sb, 1 - sb
                    wait_loads(A)
                    many = jnp.zeros((16,), jnp.int32)
                    for j in range(JG):
                        dv = didx[A][pl.ds(16 * j, 16)]
                        if scatter:
                            m = (dv >= lo) & (dv < lo + R)
                            dlb[A][0, pl.ds(16 * j, 16)] = \
                                jnp.where(m, dv - lo, R)
                            many = many | jnp.where(m, 1, 0)
                        if has_ln:
                            pidx[A][pl.ds(16 * j, 16)] = jnp.maximum(dv, 0)

                    @pl.when(bi + 1 < NBM)
                    def _prefetch():
                        issue_loads(B, bi + 1)

                    if scatter:
                        lanes16 = lax.iota(jnp.int32, 16)
                        for shf in (8, 4, 2, 1):
                            many = many | many.at[lanes16 ^ shf].get(
                                mode='promise_in_bounds')
                        any_in = many[0] > 0

                        @pl.when(any_in)
                        def _work():
                            if gathers:
                                cp1 = pltpu.async_copy(xsrc_h.at[sidx[A]],
                                                       xjs[A], sem_g[A])
                                if has_ln:
                                    cp2 = pltpu.async_copy(
                                        pred_h.at[pidx[A]], prs[A], sem_g[A])
                                cp1.wait()
                                if has_ln:
                                    cp2.wait()
                                    _sc_ln_rows(xjs[A], prs[A], gv, bv, KB)
                            pltpu.sync_copy(xjs[A],
                                            aggr_sh.at[dlb[A].at[0]],
                                            add=True)
                    else:
                        cp1 = pltpu.async_copy(xsrc_h.at[sidx[A]], xjs[A],
                                               sem_g[A])
                        cp2 = pltpu.async_copy(pred_h.at[pidx[A]], prs[A],
                                               sem_g[A])
                        cp1.wait()
                        cp2.wait()
                        _sc_ln_rows(xjs[A], prs[A], gv, bv, KB)
                        ebg = base_e + bi * KB
                        pltpu.sync_copy(xjs[A], msg_h.at[pl.ds(ebg, KB)])
                return 0
            lax.fori_loop(0, NBM // 2, pair_body, 0)

            if scatter:
                plsc.subcore_barrier()
                cbase = cid * (C * R) + lo
                for n in range(SR // ZB):
                    off = sid * SR + n * ZB
                    pltpu.sync_copy(aggr_sh.at[pl.ds(off, ZB)],
                                    sums_h.at[pl.ds(cbase + off, ZB)])
                plsc.subcore_barrier()

    if scatter:
        out_type = jax.ShapeDtypeStruct((2 * C * R, HD), F32)
    else:
        out_type = jax.ShapeDtypeStruct((EPAD, HD), F32)
    kfn = pl.kernel(body, out_type=out_type, mesh=mesh, scratch_types=scratch)
    args = []
    if not const_rows:
        args.append(xsrc)
    if has_ln:
        args += [pred, gvec, bvec]
    if gathers:
        args.append(src)
    args.append(dst)
    if const_rows:
        args.append(jnp.pad(jnp.ones((KB, 1), F32), ((0, 0), (0, HD - 1))))
    if scatter:
        args.append(jnp.zeros((ZB, HD), F32))
        return kfn(*args).reshape(2, C * R, HD)
    return kfn(*args)


def _full(shape):
    return pl.BlockSpec(shape, lambda i: tuple(0 for _ in shape))


def _mlp2_call(x, w1t, b1, w2t, b2, B=1000):
    N = x.shape[0]

    def body(x_ref, w1_ref, b1_ref, w2_ref, b2_ref, o_ref):
        h = jnp.dot(x_ref[...], w1_ref[...], preferred_element_type=F32)
        h = jnp.maximum(h + b1_ref[...], 0.0)
        o_ref[...] = jnp.dot(h, w2_ref[...],
                             preferred_element_type=F32) + b2_ref[...]

    return pl.pallas_call(
        body,
        grid=(N // B,),
        in_specs=[pl.BlockSpec((B, HD), lambda i: (i, 0)),
                  _full((HD, HD)), _full((1, HD)),
                  _full((HD, HD)), _full((1, HD))],
        out_specs=pl.BlockSpec((B, HD), lambda i: (i, 0)),
        out_shape=jax.ShapeDtypeStruct((N, HD), F32),
    )(x, w1t, b1[None], w2t, b2[None])


def _prmp_combine_call(x_dst, sums, cnts, wxt, wat, b, B=1000):
    N = x_dst.shape[0]
    G = N // B

    def body(x_ref, s0, s1, c0, c1, wx, wa, b_ref, o_ref, ps, pq):
        cnt = jnp.maximum(c0[0][:, :1] + c1[0][:, :1], 1.0)
        aggr = (s0[0] + s1[0]) / cnt
        y = jnp.dot(x_ref[...], wx[...], preferred_element_type=F32)
        y = y + jnp.dot(aggr, wa[...], preferred_element_type=F32) + b_ref[...]
        o_ref[...] = y
        ps[...] = jnp.sum(y.reshape(8, B // 8, HD), axis=1)[None]
        pq[...] = jnp.sum((y * y).reshape(8, B // 8, HD), axis=1)[None]

    sspec0 = pl.BlockSpec((1, B, HD), lambda i: (0, i, 0))
    sspec1 = pl.BlockSpec((1, B, HD), lambda i: (1, i, 0))
    cspec0 = pl.BlockSpec((1, B, HD), lambda i: (0, i, 0))
    cspec1 = pl.BlockSpec((1, B, HD), lambda i: (1, i, 0))
    return pl.pallas_call(
        body,
        grid=(G,),
        in_specs=[pl.BlockSpec((B, HD), lambda i: (i, 0)),
                  sspec0, sspec1, cspec0, cspec1,
                  _full((HD, HD)), _full((HD, HD)), _full((1, HD))],
        out_specs=[pl.BlockSpec((B, HD), lambda i: (i, 0)),
                   pl.BlockSpec((1, 8, HD), lambda i: (i, 0, 0)),
                   pl.BlockSpec((1, 8, HD), lambda i: (i, 0, 0))],
        out_shape=[jax.ShapeDtypeStruct((N, HD), F32),
                   jax.ShapeDtypeStruct((G, 8, HD), F32),
                   jax.ShapeDtypeStruct((G, 8, HD), F32)],
    )(x_dst, sums, sums, cnts, cnts, wxt, wat, b[None])


def _sage_combine_call(x_rev, sums_h, cnts_h, sums_w, cnts_w,
                       wlh_t, wlw_t, wrs_t, bls, nh, nw, B=1000):
    N = x_rev.shape[0]
    G = N // B
    gh, gw = nh // B, nw // B

    def body(x_ref, s0h, s1h, c0h, c1h, s0w, s1w, c0w, c1w,
             wlh, wlw, wrs, b_ref, o_ref, ps, pq):
        i = pl.program_id(0)
        y = jnp.dot(x_ref[...], wrs[...], preferred_element_type=F32)
        o_ref[...] = y + b_ref[...]

        @pl.when(i < gh)
        def _add_has():
            cnt = jnp.maximum(c0h[0][:, :1] + c1h[0][:, :1], 1.0)
            aggr = (s0h[0] + s1h[0]) / cnt
            o_ref[...] += jnp.dot(aggr, wlh[...], preferred_element_type=F32)

        @pl.when(i < gw)
        def _add_wrote():
            cnt = jnp.maximum(c0w[0][:, :1] + c1w[0][:, :1], 1.0)
            aggr = (s0w[0] + s1w[0]) / cnt
            o_ref[...] += jnp.dot(aggr, wlw[...], preferred_element_type=F32)

        t = o_ref[...]
        ps[...] = jnp.sum(t.reshape(8, B // 8, HD), axis=1)[None]
        pq[...] = jnp.sum((t * t).reshape(8, B // 8, HD), axis=1)[None]

    def clamp_map(g, lead):
        return lambda i: (lead, jnp.minimum(i, g - 1), 0)

    return pl.pallas_call(
        body,
        grid=(G,),
        in_specs=[pl.BlockSpec((B, HD), lambda i: (i, 0)),
                  pl.BlockSpec((1, B, HD), clamp_map(gh, 0)),
                  pl.BlockSpec((1, B, HD), clamp_map(gh, 1)),
                  pl.BlockSpec((1, B, HD), clamp_map(gh, 0)),
                  pl.BlockSpec((1, B, HD), clamp_map(gh, 1)),
                  pl.BlockSpec((1, B, HD), clamp_map(gw, 0)),
                  pl.BlockSpec((1, B, HD), clamp_map(gw, 1)),
                  pl.BlockSpec((1, B, HD), clamp_map(gw, 0)),
                  pl.BlockSpec((1, B, HD), clamp_map(gw, 1)),
                  _full((HD, HD)), _full((HD, HD)), _full((HD, HD)),
                  _full((1, HD))],
        out_specs=[pl.BlockSpec((B, HD), lambda i: (i, 0)),
                   pl.BlockSpec((1, 8, HD), lambda i: (i, 0, 0)),
                   pl.BlockSpec((1, 8, HD), lambda i: (i, 0, 0))],
        out_shape=[jax.ShapeDtypeStruct((N, HD), F32),
                   jax.ShapeDtypeStruct((G, 8, HD), F32),
                   jax.ShapeDtypeStruct((G, 8, HD), F32)],
    )(x_rev, sums_h, sums_h, cnts_h, cnts_h, sums_w, sums_w, cnts_w, cnts_w,
      wlh_t, wlw_t, wrs_t, bls[None])


def _ln_apply_call(y, ps, pq, g, b, resid, B=1000):
    N = y.shape[0]
    G = ps.shape[0]
    cnt = float(N * HD)

    def body(*refs):
        if resid is not None:
            y_ref, ps_ref, pq_ref, g_ref, b_ref, r_ref, o_ref = refs
        else:
            y_ref, ps_ref, pq_ref, g_ref, b_ref, o_ref = refs
        tot = jnp.sum(ps_ref[...])
        tsq = jnp.sum(pq_ref[...])
        mu = tot / cnt
        var = tsq / cnt - mu * mu
        denom = jnp.sqrt(jnp.maximum(var, 0.0)) + 1e-5
        out = jnp.maximum((y_ref[...] - mu) / denom * g_ref[...] + b_ref[...],
                          0.0)
        if resid is not None:
            out = out + r_ref[...]
        o_ref[...] = out

    in_specs = [pl.BlockSpec((B, HD), lambda i: (i, 0)),
                _full((G, 8, HD)), _full((G, 8, HD)),
                _full((1, HD)), _full((1, HD))]
    args = [y, ps, pq, g[None], b[None]]
    if resid is not None:
        in_specs.append(pl.BlockSpec((B, HD), lambda i: (i, 0)))
        args.append(resid)
    return pl.pallas_call(
        body,
        grid=(N // B,),
        in_specs=in_specs,
        out_specs=pl.BlockSpec((B, HD), lambda i: (i, 0)),
        out_shape=jax.ShapeDtypeStruct((N, HD), F32),
    )(*args)


def _head_call(x, w1t, b1, w2, b2, B=1000):
    N = x.shape[0]

    def body(x_ref, w1_ref, b1_ref, w2_ref, b2_ref, o_ref):
        h = jnp.dot(x_ref[...], w1_ref[...], preferred_element_type=F32)
        h = jnp.maximum(h + b1_ref[...], 0.0)
        o_ref[...] = jnp.dot(h, w2_ref[...],
                             preferred_element_type=F32) + b2_ref[...]

    return pl.pallas_call(
        body,
        grid=(N // B,),
        in_specs=[pl.BlockSpec((B, HD), lambda i: (i, 0)),
                  _full((HD, 64)), _full((1, 64)),
                  _full((64, 1)), _full((1, 1))],
        out_specs=pl.BlockSpec((B, 1), lambda i: (i, 0)),
        out_shape=jax.ShapeDtypeStruct((N, 1), F32),
    )(x, w1t, b1[None], w2, b2[None])


def _pad_edges(ei):
    pad = EPAD - E
    src = jnp.concatenate([ei[0], jnp.zeros((pad,), jnp.int32)])
    dst = jnp.concatenate([ei[1], jnp.full((pad,), -1, jnp.int32)])
    return src, dst


def _sort_ei(ei):
    perm = jnp.argsort(ei[1])
    return jnp.stack([ei[0][perm], ei[1][perm]])


def _mlp2_p(p, x):
    return _mlp2_call(x, p['l1']['W'].T, p['l1']['b'],
                      p['l2']['W'].T, p['l2']['b'])


def kernel(params, x_product, x_customer, x_review, ei_of_product,
           ei_has_review, ei_by_customer, ei_wrote_review):
    p = params
    hp = _mlp2_p(p['enc_product'], x_product)
    hc = _mlp2_p(p['enc_customer'], x_customer)
    hr = _mlp2_p(p['enc_review'], x_review)

    src_op, dst_op = _pad_edges(ei_of_product)
    src_hr, dst_hr = _pad_edges(ei_has_review)
    src_bc, dst_bc = _pad_edges(_sort_ei(ei_by_customer))
    src_wr, dst_wr = _pad_edges(_sort_ei(ei_wrote_review))

    RA = 10240
    CP, CC = 1, 5

    c_op = _sc_segment_call('count', None, None, None, None, None, dst_op,
                            CP, RA)
    c_bc = _sc_segment_call('count', None, None, None, None, None, dst_bc,
                            CC, RA)
    c_hr = _sc_segment_call('count', None, None, None, None, None, dst_hr,
                            CP, RA)
    c_wr = _sc_segment_call('count', None, None, None, None, None, dst_wr,
                            CC, RA)

    for li in range(2):
        L = p['layers'][li]
        pp, pc = L['prmp_product'], L['prmp_customer']
        pred_p = _mlp2_p(pp['pred'], hp)
        pred_c = _mlp2_p(pc['pred'], hc)

        s_pp = _sc_segment_call('gather_ln', hr, pred_p, pp['norm_g'],
                                pp['norm_b'], src_op, dst_op, CP, RA)
        msg_c = _sc_segment_call('msg', hr, pred_c, pc['norm_g'],
                                 pc['norm_b'], src_bc, dst_bc, 1, RA)
        s_pc = _sc_segment_call('linear', msg_c, None, None, None,
                                None, dst_bc, CC, RA)
        s_sh = _sc_segment_call('gather', hp, None, None, None,
                                src_hr, dst_hr, CP, RA)
        s_sw = _sc_segment_call('gather', hc, None, None, None,
                                src_wr, dst_wr, CC, RA)
        c_pp, c_pc, c_sh, c_sw = c_op, c_bc, c_hr, c_wr

        Wp = pp['update']['W']
        out_p, psp, pqp = _prmp_combine_call(
            hp, s_pp, c_pp, Wp[:, :HD].T, Wp[:, HD:].T, pp['update']['b'])
        Wc = pc['update']['W']
        out_c, psc, pqc = _prmp_combine_call(
            hc, s_pc, c_pc, Wc[:, :HD].T, Wc[:, HD:].T, pc['update']['b'])

        sh, sw = L['sage_has'], L['sage_wrote']
        out_r, psr, pqr = _sage_combine_call(
            hr, s_sh, c_sh, s_sw, c_sw,
            sh['Wl'].T, sw['Wl'].T, (sh['Wr'] + sw['Wr']).T,
            sh['bl'] + sw['bl'], 10000, 50000)

        nrm = L['norm']
        res = li > 0
        hp_n = _ln_apply_call(out_p, psp, pqp, nrm['product']['g'],
                              nrm['product']['b'], hp if res else None)
        hc_n = _ln_apply_call(out_c, psc, pqc, nrm['customer']['g'],
                              nrm['customer']['b'], hc if res else None)
        hr_n = _ln_apply_call(out_r, psr, pqr, nrm['review']['g'],
                              nrm['review']['b'], hr if res else None)
        hp, hc, hr = hp_n, hc_n, hr_n

    out = _head_call(hr, p['head1']['W'].T, p['head1']['b'],
                     p['head2']['W'].T, p['head2']['b'])
    return out[:, 0]
